# baseline (speedup 1.0000x reference)
import functools

import jax
import jax.numpy as jnp
from jax import lax
from jax.experimental import pallas as pl
from jax.experimental.pallas import tpu as pltpu
from jax.experimental.pallas import tpu_sc as plsc

_T, _D, _H, _E, _K = 2048, 1024, 4096, 8, 2
_TM = 512
_MAXT = _T * _K // _TM + _E - 1
_NTE = 16
_SPAD = _MAXT * _TM
_HBLK = 512
_NHB = _H // _HBLK
_C = 256

_NC, _NS = 2, 16
_NW = _NC * _NS


def _router_body(x_ref, wr_ref, br_ref, d0_ref, d1_ref, te_ref, nt_ref):
    x = x_ref[...]
    logits = jnp.dot(x, wr_ref[...], preferred_element_type=jnp.float32)
    logits = logits + br_ref[...]
    col = lax.broadcasted_iota(jnp.int32, (_T, _E), 1)
    rank = jnp.zeros((_T, _E), jnp.int32)
    for ep in range(_E):
        le = logits[:, ep:ep + 1]
        rank = rank + (le > logits).astype(jnp.int32)
        rank = rank + ((le == logits) & (ep < col)).astype(jnp.int32)
    oh0 = (rank == 0).astype(jnp.float32)
    oh1 = (rank == 1).astype(jnp.float32)

    counts = (jnp.sum(oh0, axis=0, keepdims=True)
              + jnp.sum(oh1, axis=0, keepdims=True))
    nt = jnp.floor((counts + float(_TM - 1)) * (1.0 / _TM))
    erow = lax.broadcasted_iota(jnp.int32, (_E, _E), 0)
    ecol = lax.broadcasted_iota(jnp.int32, (_E, _E), 1)
    ustrict = (erow < ecol).astype(jnp.float32)
    offt = jnp.dot(nt, ustrict, preferred_element_type=jnp.float32)
    offrow = offt * float(_TM)

    ci = lax.broadcasted_iota(jnp.int32, (_C, _C), 0)
    cj = lax.broadcasted_iota(jnp.int32, (_C, _C), 1)
    lstrict = (cj < ci).astype(jnp.float32)
    carry = jnp.zeros((1, _E), jnp.float32)
    for oh, dref in ((oh0, d0_ref), (oh1, d1_ref)):
        for cix in range(_T // _C):
            ohc = oh[cix * _C:(cix + 1) * _C, :]
            rk = jnp.dot(lstrict, ohc, preferred_element_type=jnp.float32)
            rk = rk + carry
            dvals = jnp.sum((rk + offrow) * ohc, axis=1, keepdims=True)
            dref[cix * _C:(cix + 1) * _C, :] = dvals.astype(jnp.int32)
            carry = carry + jnp.sum(ohc, axis=0, keepdims=True)

    ends = offt + nt
    jrow = lax.broadcasted_iota(jnp.int32, (_NTE, _E), 0).astype(jnp.float32)
    te = jnp.sum((jrow >= ends).astype(jnp.float32), axis=1, keepdims=True)
    eidx = lax.broadcasted_iota(jnp.int32, (1, _E), 1).astype(jnp.float32)
    te_last = jnp.max((counts > 0.0).astype(jnp.float32) * eidx)
    te_ref[...] = jnp.where(te < float(_E), te, te_last).astype(jnp.int32)
    nt_ref[...] = jnp.sum(nt, axis=1, keepdims=True).astype(jnp.int32)


def _router(xf, Wr, br2):
    return pl.pallas_call(
        _router_body,
        in_specs=[
            pl.BlockSpec((_T, _D), lambda: (0, 0)),
            pl.BlockSpec((_D, _E), lambda: (0, 0)),
            pl.BlockSpec((1, _E), lambda: (0, 0)),
        ],
        out_specs=(
            pl.BlockSpec((_T, 1), lambda: (0, 0)),
            pl.BlockSpec((_T, 1), lambda: (0, 0)),
            pl.BlockSpec((_NTE, 1), lambda: (0, 0)),
            pl.BlockSpec((1, 1), lambda: (0, 0)),
        ),
        out_shape=(
            jax.ShapeDtypeStruct((_T, 1), jnp.int32),
            jax.ShapeDtypeStruct((_T, 1), jnp.int32),
            jax.ShapeDtypeStruct((_NTE, 1), jnp.int32),
            jax.ShapeDtypeStruct((1, 1), jnp.int32),
        ),
    )(xf, Wr, br2)


_TPW = _T // _NW
_SCCH = 32

_sc_mesh = plsc.VectorSubcoreMesh(core_axis_name="c", subcore_axis_name="s",
                                  num_cores=_NC, num_subcores=_NS)


@functools.partial(
    pl.kernel,
    out_type=jax.ShapeDtypeStruct((_SPAD, _D), jnp.float32),
    mesh=_sc_mesh,
    scratch_types=[
        pltpu.VMEM((2, _SCCH, _D), jnp.float32),
        pltpu.VMEM((2, _SCCH), jnp.int32),
        pltpu.VMEM((2, _SCCH), jnp.int32),
        pltpu.SemaphoreType.DMA,
        pltpu.SemaphoreType.DMA,
        pltpu.SemaphoreType.DMA,
    ],
)
def _sc_scatter(x_hbm, d0_hbm, d1_hbm, xs_hbm, xbuf, idx0, idx1,
                semx, sem0, sem1):
    wid = lax.axis_index("s") * _NC + lax.axis_index("c")
    nch = _TPW // _SCCH
    base = wid * _TPW
    pltpu.sync_copy(d0_hbm.at[pl.ds(base, _SCCH)], idx0.at[0])
    pltpu.sync_copy(d1_hbm.at[pl.ds(base, _SCCH)], idx1.at[0])
    ld = pltpu.async_copy(x_hbm.at[pl.ds(base, _SCCH)], xbuf.at[0], semx)
    for ch in range(nch):
        t1 = base + (ch + 1) * _SCCH
        ld.wait()
        if ch + 1 < nch:
            pltpu.sync_copy(d0_hbm.at[pl.ds(t1, _SCCH)], idx0.at[(ch + 1) % 2])
            pltpu.sync_copy(d1_hbm.at[pl.ds(t1, _SCCH)], idx1.at[(ch + 1) % 2])
            ld = pltpu.async_copy(x_hbm.at[pl.ds(t1, _SCCH)],
                                  xbuf.at[(ch + 1) % 2], semx)
        cp0 = pltpu.async_copy(xbuf.at[ch % 2], xs_hbm.at[idx0.at[ch % 2]],
                               sem0)
        cp1 = pltpu.async_copy(xbuf.at[ch % 2], xs_hbm.at[idx1.at[ch % 2]],
                               sem1)
        cp0.wait()
        cp1.wait()


_DHALF = _D // 2
_HHALF = _H // 2


def _gemm_body(te_ref, nt_ref, xs_ref, w1_ref, b1_ref, w2_ref, b2_ref,
               out_ref, h_ref, acc_ref):
    j = pl.program_id(0)
    p = pl.program_id(1)

    @pl.when(j < nt_ref[0])
    def _compute():
        @pl.when(p == 0)
        def _p0():
            h_ref[...] = jnp.dot(xs_ref[...], w1_ref[0],
                                 preferred_element_type=jnp.float32)

        @pl.when(p == 1)
        def _p1():
            hfull = h_ref[...] + jnp.dot(xs_ref[...], w1_ref[0],
                                         preferred_element_type=jnp.float32)
            h_ref[...] = jnp.maximum(hfull + b1_ref[0], 0.0)

        @pl.when(p == 2)
        def _p2():
            acc_ref[...] = jnp.dot(h_ref[:, :_HHALF], w2_ref[0],
                                   preferred_element_type=jnp.float32)

        @pl.when(p == 3)
        def _p3():
            y = acc_ref[...] + jnp.dot(h_ref[:, _HHALF:], w2_ref[0],
                                       preferred_element_type=jnp.float32)
            out_ref[...] = y * (1.0 / _K) + (1.0 / _K) * b2_ref[0]


def _gemm(te_arr, nt_arr, xs, W1b, b1r, W2b, b2r):
    grid_spec = pltpu.PrefetchScalarGridSpec(
        num_scalar_prefetch=2,
        grid=(_MAXT, 4),
        in_specs=[
            pl.BlockSpec(
                (_TM, _DHALF),
                lambda j, hb, te, nt: (jnp.minimum(j, nt[0] - 1),
                                       jnp.minimum(hb, 1))),
            pl.BlockSpec((1, _DHALF, _H),
                         lambda j, hb, te, nt: (te[j], jnp.minimum(hb, 1), 0)),
            pl.BlockSpec((1, 1, _H), lambda j, hb, te, nt: (te[j], 0, 0)),
            pl.BlockSpec((1, _HHALF, _D),
                         lambda j, hb, te, nt: (te[j], jnp.maximum(hb - 2, 0),
                                                0)),
            pl.BlockSpec((1, 1, _D), lambda j, hb, te, nt: (te[j], 0, 0)),
        ],
        out_specs=pl.BlockSpec((_TM, _D), lambda j, hb, te, nt: (j, 0)),
        scratch_shapes=[pltpu.VMEM((_TM, _H), jnp.float32),
                        pltpu.VMEM((_TM, _D), jnp.float32)],
    )
    return pl.pallas_call(
        _gemm_body,
        grid_spec=grid_spec,
        out_shape=jax.ShapeDtypeStruct((_SPAD, _D), jnp.float32),
        compiler_params=pltpu.CompilerParams(
            dimension_semantics=("arbitrary", "arbitrary"),
            vmem_limit_bytes=62 * 1024 * 1024),
    )(te_arr, nt_arr, xs, W1b, b1r, W2b, b2r)


_CMCH = 16


@functools.partial(
    pl.kernel,
    out_type=jax.ShapeDtypeStruct((_T, _D), jnp.float32),
    mesh=_sc_mesh,
    scratch_types=[
        pltpu.VMEM((2, _CMCH, _D), jnp.float32),
        pltpu.VMEM((2, _CMCH, _D), jnp.float32),
        pltpu.VMEM((2, _CMCH), jnp.int32),
        pltpu.VMEM((2, _CMCH), jnp.int32),
        pltpu.SemaphoreType.DMA,
        pltpu.SemaphoreType.DMA,
    ],
)
def _sc_combine(ys_hbm, d0_hbm, d1_hbm, out_hbm,
                bufa, bufb, idx0, idx1, sem0, sem1):
    wid = lax.axis_index("s") * _NC + lax.axis_index("c")
    nch = _TPW // _CMCH
    base = wid * _TPW
    pltpu.sync_copy(d0_hbm.at[pl.ds(base, _CMCH)], idx0.at[0])
    pltpu.sync_copy(d1_hbm.at[pl.ds(base, _CMCH)], idx1.at[0])
    cp0 = pltpu.async_copy(ys_hbm.at[idx0.at[0]], bufa.at[0], sem0)
    cp1 = pltpu.async_copy(ys_hbm.at[idx1.at[0]], bufb.at[0], sem1)
    for ch in range(nch):
        t0 = base + ch * _CMCH
        t1 = t0 + _CMCH
        cp0.wait()
        cp1.wait()
        if ch + 1 < nch:
            nxt = (ch + 1) % 2
            pltpu.sync_copy(d0_hbm.at[pl.ds(t1, _CMCH)], idx0.at[nxt])
            pltpu.sync_copy(d1_hbm.at[pl.ds(t1, _CMCH)], idx1.at[nxt])
            cp0 = pltpu.async_copy(ys_hbm.at[idx0.at[nxt]], bufa.at[nxt], sem0)
            cp1 = pltpu.async_copy(ys_hbm.at[idx1.at[nxt]], bufb.at[nxt], sem1)
        cur = ch % 2

        def _row(i, c):
            for jj in range(_D // 16):
                sl = pl.ds(jj * 16, 16)
                bufa[cur, i, sl] = bufa[cur, i, sl] + bufb[cur, i, sl]
            return c

        lax.fori_loop(0, _CMCH, _row, 0)
        pltpu.sync_copy(bufa.at[cur], out_hbm.at[pl.ds(t0, _CMCH)])


def kernel(x, Wr, br, W1, b1, W2, b2):
    B_, S_, D_ = x.shape
    xf = x.reshape(S_, D_)
    br2 = br.reshape(1, _E)
    b1r = b1.reshape(_E, 1, _H)
    b2r = b2.reshape(_E, 1, _D)

    d0_2d, d1_2d, te_2d, nt_2d = _router(xf, Wr, br2)
    d0 = d0_2d.reshape(S_)
    d1 = d1_2d.reshape(S_)
    te_arr = te_2d.reshape(_NTE)
    nt_arr = nt_2d.reshape(1)

    xs = _sc_scatter(xf, d0, d1)
    ys = _gemm(te_arr, nt_arr, xs, W1, b1r, W2, b2r)
    out = _sc_combine(ys, d0, d1)
    return out.reshape(B_, S_, D_)


# speedup vs baseline: 1.2162x; 1.2162x over previous
import functools

import jax
import jax.numpy as jnp
from jax import lax
from jax.experimental import pallas as pl
from jax.experimental.pallas import tpu as pltpu
from jax.experimental.pallas import tpu_sc as plsc

_T, _D, _H, _E, _K = 2048, 1024, 4096, 8, 2
_TM = 512
_MAXT = _T * _K // _TM + _E - 1
_NTE = 16
_SPAD = _MAXT * _TM
_HBLK = 512
_NHB = _H // _HBLK
_C = 256

_NC, _NS = 2, 16
_NW = _NC * _NS


def _router_body(x_ref, wr_ref, br_ref, d0_ref, d1_ref, te_ref, nt_ref):
    x = x_ref[...]
    logits = jnp.dot(x, wr_ref[...], preferred_element_type=jnp.float32)
    logits = logits + br_ref[...]
    col = lax.broadcasted_iota(jnp.int32, (_T, _E), 1)
    rank = jnp.zeros((_T, _E), jnp.int32)
    for ep in range(_E):
        le = logits[:, ep:ep + 1]
        rank = rank + (le > logits).astype(jnp.int32)
        rank = rank + ((le == logits) & (ep < col)).astype(jnp.int32)
    oh0 = (rank == 0).astype(jnp.float32)
    oh1 = (rank == 1).astype(jnp.float32)

    counts = (jnp.sum(oh0, axis=0, keepdims=True)
              + jnp.sum(oh1, axis=0, keepdims=True))
    nt = jnp.floor((counts + float(_TM - 1)) * (1.0 / _TM))
    erow = lax.broadcasted_iota(jnp.int32, (_E, _E), 0)
    ecol = lax.broadcasted_iota(jnp.int32, (_E, _E), 1)
    ustrict = (erow < ecol).astype(jnp.float32)
    offt = jnp.dot(nt, ustrict, preferred_element_type=jnp.float32)
    offrow = offt * float(_TM)

    ci = lax.broadcasted_iota(jnp.int32, (_C, _C), 0)
    cj = lax.broadcasted_iota(jnp.int32, (_C, _C), 1)
    lstrict = (cj < ci).astype(jnp.float32)
    carry = jnp.zeros((1, _E), jnp.float32)
    for oh, dref in ((oh0, d0_ref), (oh1, d1_ref)):
        for cix in range(_T // _C):
            ohc = oh[cix * _C:(cix + 1) * _C, :]
            rk = jnp.dot(lstrict, ohc, preferred_element_type=jnp.float32)
            rk = rk + carry
            dvals = jnp.sum((rk + offrow) * ohc, axis=1, keepdims=True)
            dref[cix * _C:(cix + 1) * _C, :] = dvals.astype(jnp.int32)
            carry = carry + jnp.sum(ohc, axis=0, keepdims=True)

    ends = offt + nt
    jrow = lax.broadcasted_iota(jnp.int32, (_NTE, _E), 0).astype(jnp.float32)
    te = jnp.sum((jrow >= ends).astype(jnp.float32), axis=1, keepdims=True)
    eidx = lax.broadcasted_iota(jnp.int32, (1, _E), 1).astype(jnp.float32)
    te_last = jnp.max((counts > 0.0).astype(jnp.float32) * eidx)
    te_ref[...] = jnp.where(te < float(_E), te, te_last).astype(jnp.int32)
    nt_ref[...] = jnp.sum(nt, axis=1, keepdims=True).astype(jnp.int32)


def _router(xf, Wr, br2):
    return pl.pallas_call(
        _router_body,
        in_specs=[
            pl.BlockSpec((_T, _D), lambda: (0, 0)),
            pl.BlockSpec((_D, _E), lambda: (0, 0)),
            pl.BlockSpec((1, _E), lambda: (0, 0)),
        ],
        out_specs=(
            pl.BlockSpec((_T, 1), lambda: (0, 0)),
            pl.BlockSpec((_T, 1), lambda: (0, 0)),
            pl.BlockSpec((_NTE, 1), lambda: (0, 0)),
            pl.BlockSpec((1, 1), lambda: (0, 0)),
        ),
        out_shape=(
            jax.ShapeDtypeStruct((_T, 1), jnp.int32),
            jax.ShapeDtypeStruct((_T, 1), jnp.int32),
            jax.ShapeDtypeStruct((_NTE, 1), jnp.int32),
            jax.ShapeDtypeStruct((1, 1), jnp.int32),
        ),
    )(xf, Wr, br2)


_TPW = _T // _NW
_SCCH = 32

_sc_mesh = plsc.VectorSubcoreMesh(core_axis_name="c", subcore_axis_name="s",
                                  num_cores=_NC, num_subcores=_NS)


@functools.partial(
    pl.kernel,
    out_type=jax.ShapeDtypeStruct((_SPAD, _D), jnp.float32),
    mesh=_sc_mesh,
    scratch_types=[
        pltpu.VMEM((2, _SCCH, _D), jnp.float32),
        pltpu.VMEM((2, _SCCH), jnp.int32),
        pltpu.VMEM((2, _SCCH), jnp.int32),
        pltpu.SemaphoreType.DMA,
        pltpu.SemaphoreType.DMA,
        pltpu.SemaphoreType.DMA,
    ],
)
def _sc_scatter(x_hbm, d0_hbm, d1_hbm, xs_hbm, xbuf, idx0, idx1,
                semx, sem0, sem1):
    wid = lax.axis_index("s") * _NC + lax.axis_index("c")
    nch = _TPW // _SCCH
    base = wid * _TPW
    pltpu.sync_copy(d0_hbm.at[pl.ds(base, _SCCH)], idx0.at[0])
    pltpu.sync_copy(d1_hbm.at[pl.ds(base, _SCCH)], idx1.at[0])
    ld = pltpu.async_copy(x_hbm.at[pl.ds(base, _SCCH)], xbuf.at[0], semx)
    for ch in range(nch):
        t1 = base + (ch + 1) * _SCCH
        ld.wait()
        if ch + 1 < nch:
            pltpu.sync_copy(d0_hbm.at[pl.ds(t1, _SCCH)], idx0.at[(ch + 1) % 2])
            pltpu.sync_copy(d1_hbm.at[pl.ds(t1, _SCCH)], idx1.at[(ch + 1) % 2])
            ld = pltpu.async_copy(x_hbm.at[pl.ds(t1, _SCCH)],
                                  xbuf.at[(ch + 1) % 2], semx)
        cp0 = pltpu.async_copy(xbuf.at[ch % 2], xs_hbm.at[idx0.at[ch % 2]],
                               sem0)
        cp1 = pltpu.async_copy(xbuf.at[ch % 2], xs_hbm.at[idx1.at[ch % 2]],
                               sem1)
        cp0.wait()
        cp1.wait()


_HHALF = _H // 2


def _serp(j, hb):
    return jnp.where(j % 2 == 0, hb, 1 - hb)


def _gemm_body(te_ref, nt_ref, xs_ref, w1_ref, b1_ref, w2_ref, b2_ref,
               out_ref, acc_ref):
    j = pl.program_id(0)
    hb = pl.program_id(1)

    @pl.when(j < nt_ref[0])
    def _compute():
        h = jnp.maximum(
            jnp.dot(xs_ref[...], w1_ref[0], preferred_element_type=jnp.float32)
            + b1_ref[0], 0.0)
        part = jnp.dot(h, w2_ref[0], preferred_element_type=jnp.float32)
        part = part * (1.0 / _K)

        @pl.when(hb == 0)
        def _init():
            acc_ref[...] = part + (1.0 / _K) * b2_ref[0]

        @pl.when(hb == 1)
        def _emit():
            out_ref[...] = acc_ref[...] + part


def _gemm(te_arr, nt_arr, xs, W1b, b1r, W2b, b2r):
    grid_spec = pltpu.PrefetchScalarGridSpec(
        num_scalar_prefetch=2,
        grid=(_MAXT, 2),
        in_specs=[
            pl.BlockSpec((_TM, _D),
                         lambda j, hb, te, nt: (jnp.minimum(j, nt[0] - 1), 0)),
            pl.BlockSpec((1, _D, _HHALF),
                         lambda j, hb, te, nt: (te[j], 0, _serp(j, hb))),
            pl.BlockSpec((1, 1, _HHALF),
                         lambda j, hb, te, nt: (te[j], 0, _serp(j, hb))),
            pl.BlockSpec((1, _HHALF, _D),
                         lambda j, hb, te, nt: (te[j], _serp(j, hb), 0)),
            pl.BlockSpec((1, 1, _D), lambda j, hb, te, nt: (te[j], 0, 0)),
        ],
        out_specs=pl.BlockSpec((_TM, _D), lambda j, hb, te, nt: (j, 0)),
        scratch_shapes=[pltpu.VMEM((_TM, _D), jnp.float32)],
    )
    return pl.pallas_call(
        _gemm_body,
        grid_spec=grid_spec,
        out_shape=jax.ShapeDtypeStruct((_SPAD, _D), jnp.float32),
        compiler_params=pltpu.CompilerParams(
            dimension_semantics=("arbitrary", "arbitrary"),
            vmem_limit_bytes=62 * 1024 * 1024),
    )(te_arr, nt_arr, xs, W1b, b1r, W2b, b2r)


_CMCH = 16


@functools.partial(
    pl.kernel,
    out_type=jax.ShapeDtypeStruct((_T, _D), jnp.float32),
    mesh=_sc_mesh,
    scratch_types=[
        pltpu.VMEM((2, _CMCH, _D), jnp.float32),
        pltpu.VMEM((2, _CMCH, _D), jnp.float32),
        pltpu.VMEM((2, _CMCH), jnp.int32),
        pltpu.VMEM((2, _CMCH), jnp.int32),
        pltpu.SemaphoreType.DMA,
        pltpu.SemaphoreType.DMA,
    ],
)
def _sc_combine(ys_hbm, d0_hbm, d1_hbm, out_hbm,
                bufa, bufb, idx0, idx1, sem0, sem1):
    wid = lax.axis_index("s") * _NC + lax.axis_index("c")
    nch = _TPW // _CMCH
    base = wid * _TPW
    pltpu.sync_copy(d0_hbm.at[pl.ds(base, _CMCH)], idx0.at[0])
    pltpu.sync_copy(d1_hbm.at[pl.ds(base, _CMCH)], idx1.at[0])
    cp0 = pltpu.async_copy(ys_hbm.at[idx0.at[0]], bufa.at[0], sem0)
    cp1 = pltpu.async_copy(ys_hbm.at[idx1.at[0]], bufb.at[0], sem1)
    for ch in range(nch):
        t0 = base + ch * _CMCH
        t1 = t0 + _CMCH
        cp0.wait()
        cp1.wait()
        if ch + 1 < nch:
            nxt = (ch + 1) % 2
            pltpu.sync_copy(d0_hbm.at[pl.ds(t1, _CMCH)], idx0.at[nxt])
            pltpu.sync_copy(d1_hbm.at[pl.ds(t1, _CMCH)], idx1.at[nxt])
            cp0 = pltpu.async_copy(ys_hbm.at[idx0.at[nxt]], bufa.at[nxt], sem0)
            cp1 = pltpu.async_copy(ys_hbm.at[idx1.at[nxt]], bufb.at[nxt], sem1)
        cur = ch % 2

        def _row(i, c):
            for jj in range(_D // 16):
                sl = pl.ds(jj * 16, 16)
                bufa[cur, i, sl] = bufa[cur, i, sl] + bufb[cur, i, sl]
            return c

        lax.fori_loop(0, _CMCH, _row, 0)
        pltpu.sync_copy(bufa.at[cur], out_hbm.at[pl.ds(t0, _CMCH)])


def kernel(x, Wr, br, W1, b1, W2, b2):
    B_, S_, D_ = x.shape
    xf = x.reshape(S_, D_)
    br2 = br.reshape(1, _E)
    b1r = b1.reshape(_E, 1, _H)
    b2r = b2.reshape(_E, 1, _D)

    d0_2d, d1_2d, te_2d, nt_2d = _router(xf, Wr, br2)
    d0 = d0_2d.reshape(S_)
    d1 = d1_2d.reshape(S_)
    te_arr = te_2d.reshape(_NTE)
    nt_arr = nt_2d.reshape(1)

    xs = _sc_scatter(xf, d0, d1)
    ys = _gemm(te_arr, nt_arr, xs, W1, b1r, W2, b2r)
    out = _sc_combine(ys, d0, d1)
    return out.reshape(B_, S_, D_)
